# baseline (speedup 1.0000x reference)
import functools

import jax
import jax.numpy as jnp
from jax import lax
from jax.experimental import pallas as pl
from jax.experimental.pallas import tpu as pltpu
from jax.experimental.pallas import tpu_sc as plsc

N = 10000
NP = 10240
D = 256
HD = 128
QD = 64
E = 160000
EP = 163840
CT = EP // 128
NS = 16
CPT = CT // NS
RD = 8
LAG = 4
STRIPE = NP // NS
NUM_GRAPHS = 64
R = 256
NBLK = NP // R

_mesh = plsc.VectorSubcoreMesh(core_axis_name="c", subcore_axis_name="s")


@functools.partial(
    pl.kernel,
    out_type=jax.ShapeDtypeStruct((NP,), jnp.float32),
    mesh=_mesh,
    scratch_types=[
        pltpu.VMEM((CPT, 128), jnp.int32),
        pltpu.VMEM((128,), jnp.float32),
        pltpu.VMEM((STRIPE,), jnp.float32),
        pltpu.VMEM_SHARED((NP,), jnp.float32),
    ],
)
def _deg_kernel(dst_hbm, deg_hbm, dst_v, ones_v, zb_v, sdeg):
    c = lax.axis_index("c")
    s = lax.axis_index("s")
    for k in range(8):
        ones_v[pl.ds(k * 16, 16)] = jnp.ones((16,), jnp.float32)

    def zb(i, carry):
        zb_v[pl.ds(i * 16, 16)] = jnp.zeros((16,), jnp.float32)
        return carry
    lax.fori_loop(0, STRIPE // 16, zb, 0)
    pltpu.sync_copy(zb_v, sdeg.at[pl.ds(s * STRIPE, STRIPE)])
    plsc.subcore_barrier()

    pltpu.sync_copy(dst_hbm.at[pl.ds(s * CPT, CPT)], dst_v)

    def body(j, carry):
        pltpu.sync_copy(ones_v, sdeg.at[dst_v.at[j]], add=True)
        return carry
    lax.fori_loop(0, CPT, body, 0)
    plsc.subcore_barrier()

    @pl.when(c == 0)
    def _():
        pltpu.sync_copy(sdeg.at[pl.ds(s * STRIPE, STRIPE)],
                        deg_hbm.at[pl.ds(s * STRIPE, STRIPE)])


@functools.partial(
    pl.kernel,
    out_type=jax.ShapeDtypeStruct((4, NP, QD), jnp.float32),
    mesh=_mesh,
    compiler_params=pltpu.CompilerParams(use_tc_tiling_on_sc=False),
    scratch_types=[
        pltpu.VMEM((CPT, 128), jnp.int32),
        pltpu.VMEM((CPT, 128), jnp.int32),
        pltpu.VMEM((RD, 128, QD), jnp.float32),
        pltpu.VMEM_SHARED((NP, QD), jnp.float32),
        pltpu.SemaphoreType.DMA((RD,)),
        pltpu.SemaphoreType.DMA((RD,)),
    ],
)
def _agg_kernel(sflat_hbm, src_hbm, dst_hbm, zeros_hbm, out_hbm,
                src_v, dst_v, rows, sout, gsem, ssem):
    c = lax.axis_index("c")
    s = lax.axis_index("s")

    pltpu.sync_copy(dst_hbm.at[pl.ds(s * CPT, CPT)], dst_v)

    def pass_body(p, pcarry):
        q = 2 * c + p
        pltpu.sync_copy(zeros_hbm.at[pl.ds(s * STRIPE, STRIPE)],
                        sout.at[pl.ds(s * STRIPE, STRIPE)])
        pltpu.sync_copy(src_hbm.at[q, pl.ds(s * CPT, CPT)], src_v)
        plsc.subcore_barrier()

        def body(j, carry):
            b = lax.rem(j, RD)

            @pl.when(jnp.logical_and(j >= RD, j - RD < CPT))
            def _():
                pltpu.make_async_copy(
                    rows.at[b], sout.at[dst_v.at[lax.rem(j, CPT)]],
                    ssem.at[b]).wait()

            @pl.when(j < CPT)
            def _():
                pltpu.async_copy(sflat_hbm.at[src_v.at[j]],
                                 rows.at[b], gsem.at[b])

            jj = j - LAG
            bb = lax.rem(j + RD - LAG, RD)

            @pl.when(jnp.logical_and(jj >= 0, jj < CPT))
            def _():
                pltpu.make_async_copy(sflat_hbm.at[pl.ds(0, 128)],
                                      rows.at[bb], gsem.at[bb]).wait()
                pltpu.async_copy(rows.at[bb], sout.at[dst_v.at[jj]],
                                 ssem.at[bb], add=True)
            return carry
        lax.fori_loop(0, CPT + RD, body, 0)

        plsc.subcore_barrier()
        pltpu.sync_copy(sout.at[pl.ds(s * STRIPE, STRIPE)],
                        out_hbm.at[q, pl.ds(s * STRIPE, STRIPE)])
        return pcarry
    lax.fori_loop(0, 2, pass_body, 0)


def _mm_halves(h0, h1, w_ref):
    outs = []
    for cp in range(2):
        u = jnp.dot(h0, w_ref[0, :, cp, :], preferred_element_type=jnp.float32)
        u = u + jnp.dot(h1, w_ref[1, :, cp, :],
                        preferred_element_type=jnp.float32)
        outs.append(u)
    return outs


def _store_quarters(o_ref, u0, u1, dinv):
    v0 = u0 * dinv
    v1 = u1 * dinv
    o_ref[0] = v0[:, :QD]
    o_ref[1] = v0[:, QD:]
    o_ref[2] = v1[:, :QD]
    o_ref[3] = v1[:, QD:]


def _a1_body(x_ref, deg_ref, w_ref, o_ref):
    dinv = lax.rsqrt(deg_ref[0, 0, :] + 1.0)[:, None]
    u0, u1 = _mm_halves(x_ref[:, :HD], x_ref[:, HD:], w_ref)
    _store_quarters(o_ref, u0, u1, dinv)


def _halves(q_ref):
    a = jnp.concatenate([q_ref[0], q_ref[1]], axis=1)
    b = jnp.concatenate([q_ref[2], q_ref[3]], axis=1)
    return a, b


def _a_body(e_ref, s_ref, deg_ref, w_ref, b_ref, o_ref):
    dinv = lax.rsqrt(deg_ref[0, 0, :] + 1.0)[:, None]
    e0, e1 = _halves(e_ref)
    s0, s1 = _halves(s_ref)
    h0 = jnp.maximum(dinv * (e0 + s0) + b_ref[0], 0.0)
    h1 = jnp.maximum(dinv * (e1 + s1) + b_ref[1], 0.0)
    u0, u1 = _mm_halves(h0, h1, w_ref)
    _store_quarters(o_ref, u0, u1, dinv)


def _f_body(e_ref, s_ref, deg_ref, b_ref, batch_ref,
            l1w_ref, l1b_ref, l2w_ref, l2b_ref, o_ref,
            acc0, acc1, cnt):
    i = pl.program_id(0)

    @pl.when(i == 0)
    def _():
        acc0[:] = jnp.zeros((NUM_GRAPHS, HD), jnp.float32)
        acc1[:] = jnp.zeros((NUM_GRAPHS, HD), jnp.float32)
        cnt[:] = jnp.zeros((NUM_GRAPHS, HD), jnp.float32)

    dinv = lax.rsqrt(deg_ref[0, 0, :] + 1.0)[:, None]
    e0, e1 = _halves(e_ref)
    s0, s1 = _halves(s_ref)
    h0 = jnp.maximum(dinv * (e0 + s0) + b_ref[0], 0.0)
    h1 = jnp.maximum(dinv * (e1 + s1) + b_ref[1], 0.0)
    gids = lax.broadcasted_iota(jnp.int32, (NUM_GRAPHS, R), 0)
    pt = (gids == batch_ref[0, 0, :][None, :]).astype(jnp.float32)
    acc0[:] += jnp.dot(pt, h0, preferred_element_type=jnp.float32)
    acc1[:] += jnp.dot(pt, h1, preferred_element_type=jnp.float32)
    cnt[:] += jnp.broadcast_to(jnp.sum(pt, axis=1, keepdims=True),
                               (NUM_GRAPHS, HD))

    @pl.when(i == NBLK - 1)
    def _():
        cts = jnp.maximum(cnt[:, 0:1], 1.0)
        p0 = acc0[:] / cts
        p1 = acc1[:] / cts
        z = jnp.dot(p0, l1w_ref[0], preferred_element_type=jnp.float32)
        z = z + jnp.dot(p1, l1w_ref[1], preferred_element_type=jnp.float32)
        z = jnp.maximum(z + l1b_ref[0, :][None, :], 0.0)
        o_ref[:] = (jnp.dot(z, l2w_ref[:], preferred_element_type=jnp.float32)
                    + l2b_ref[0, :][None, :])


def _tc_a1(xp, deg3, w):
    return pl.pallas_call(
        _a1_body,
        grid=(NBLK,),
        in_specs=[
            pl.BlockSpec((R, D), lambda i: (i, 0)),
            pl.BlockSpec((1, 1, R), lambda i: (i, 0, 0)),
            pl.BlockSpec((2, HD, 2, HD), lambda i: (0, 0, 0, 0)),
        ],
        out_specs=pl.BlockSpec((4, R, QD), lambda i: (0, i, 0)),
        out_shape=jax.ShapeDtypeStruct((4, NP, QD), jnp.float32),
    )(xp, deg3, w)


def _tc_a(e, sv, deg3, w, b):
    return pl.pallas_call(
        _a_body,
        grid=(NBLK,),
        in_specs=[
            pl.BlockSpec((4, R, QD), lambda i: (0, i, 0)),
            pl.BlockSpec((4, R, QD), lambda i: (0, i, 0)),
            pl.BlockSpec((1, 1, R), lambda i: (i, 0, 0)),
            pl.BlockSpec((2, HD, 2, HD), lambda i: (0, 0, 0, 0)),
            pl.BlockSpec((2, HD), lambda i: (0, 0)),
        ],
        out_specs=pl.BlockSpec((4, R, QD), lambda i: (0, i, 0)),
        out_shape=jax.ShapeDtypeStruct((4, NP, QD), jnp.float32),
    )(e, sv, deg3, w, b)


def _tc_final(e, sv, deg3, b, batch3, l1w, l1b, l2w, l2b):
    return pl.pallas_call(
        _f_body,
        grid=(NBLK,),
        in_specs=[
            pl.BlockSpec((4, R, QD), lambda i: (0, i, 0)),
            pl.BlockSpec((4, R, QD), lambda i: (0, i, 0)),
            pl.BlockSpec((1, 1, R), lambda i: (i, 0, 0)),
            pl.BlockSpec((2, HD), lambda i: (0, 0)),
            pl.BlockSpec((1, 1, R), lambda i: (i, 0, 0)),
            pl.BlockSpec((2, HD, D), lambda i: (0, 0, 0)),
            pl.BlockSpec((1, D), lambda i: (0, 0)),
            pl.BlockSpec((D, 10), lambda i: (0, 0)),
            pl.BlockSpec((1, 10), lambda i: (0, 0)),
        ],
        out_specs=pl.BlockSpec((NUM_GRAPHS, 10), lambda i: (0, 0)),
        out_shape=jax.ShapeDtypeStruct((NUM_GRAPHS, 10), jnp.float32),
        scratch_shapes=[
            pltpu.VMEM((NUM_GRAPHS, HD), jnp.float32),
            pltpu.VMEM((NUM_GRAPHS, HD), jnp.float32),
            pltpu.VMEM((NUM_GRAPHS, HD), jnp.float32),
        ],
    )(e, sv, deg3, b, batch3, l1w, l1b, l2w, l2b)


def kernel(x, edge_index, batch, conv_W, conv_b, lin1_W, lin1_b, lin2_W,
           lin2_b):
    src = edge_index[0].astype(jnp.int32)
    dst = edge_index[1].astype(jnp.int32)
    perm = jnp.argsort(src)
    src = src[perm]
    dst = dst[perm]
    pad = jnp.full((EP - E,), N, jnp.int32)
    src_p = jnp.concatenate([src, pad])
    dst_p = jnp.concatenate([dst, pad])
    src4 = jnp.stack([src_p + q * NP for q in range(4)]).reshape(4, CT, 128)
    dst2 = dst_p.reshape(CT, 128)

    xp = jnp.pad(x, ((0, NP - N), (0, 0)))
    batch3 = jnp.concatenate(
        [batch.astype(jnp.int32),
         jnp.full((NP - N,), NUM_GRAPHS, jnp.int32)]).reshape(NBLK, 1, R)

    w = conv_W.reshape(3, 2, HD, 2, HD)
    b = conv_b.reshape(3, 2, HD)
    l1w = lin1_W.reshape(2, HD, D)
    l1b = lin1_b.reshape(1, D)
    l2b = lin2_b.reshape(1, 10)

    deg = _deg_kernel(dst2)
    deg3 = deg.reshape(NBLK, 1, R)

    s1 = _tc_a1(xp, deg3, w[0])
    zeros_h = jnp.zeros((NP, QD), jnp.float32)
    e1 = _agg_kernel(s1.reshape(4 * NP, QD), src4, dst2, zeros_h)
    s2 = _tc_a(e1, s1, deg3, w[1], b[0])
    e2 = _agg_kernel(s2.reshape(4 * NP, QD), src4, dst2, zeros_h)
    s3 = _tc_a(e2, s2, deg3, w[2], b[1])
    e3 = _agg_kernel(s3.reshape(4 * NP, QD), src4, dst2, zeros_h)

    return _tc_final(e3, s3, deg3, b[2], batch3, l1w, l1b, lin2_W, l2b)


# speedup vs baseline: 1.1629x; 1.1629x over previous
import functools

import jax
import jax.numpy as jnp
from jax import lax
from jax.experimental import pallas as pl
from jax.experimental.pallas import tpu as pltpu
from jax.experimental.pallas import tpu_sc as plsc

N = 10000
NP = 10240
D = 256
HD = 128
QD = 64
E = 160000
EP = 163840
CT = EP // 128
NS = 16
CPT = CT // NS
RD = 8
LAG = 4
STRIPE = NP // NS
NUM_GRAPHS = 64
R = 256
NBLK = NP // R

_mesh = plsc.VectorSubcoreMesh(core_axis_name="c", subcore_axis_name="s")


@functools.partial(
    pl.kernel,
    out_type=jax.ShapeDtypeStruct((NP,), jnp.float32),
    mesh=_mesh,
    scratch_types=[
        pltpu.VMEM((CPT, 128), jnp.int32),
        pltpu.VMEM((128,), jnp.float32),
        pltpu.VMEM((STRIPE,), jnp.float32),
        pltpu.VMEM_SHARED((NP,), jnp.float32),
    ],
)
def _deg_kernel(dst_hbm, deg_hbm, dst_v, ones_v, zb_v, sdeg):
    c = lax.axis_index("c")
    s = lax.axis_index("s")
    for k in range(8):
        ones_v[pl.ds(k * 16, 16)] = jnp.ones((16,), jnp.float32)

    def zb(i, carry):
        zb_v[pl.ds(i * 16, 16)] = jnp.zeros((16,), jnp.float32)
        return carry
    lax.fori_loop(0, STRIPE // 16, zb, 0)
    pltpu.sync_copy(zb_v, sdeg.at[pl.ds(s * STRIPE, STRIPE)])
    plsc.subcore_barrier()

    pltpu.sync_copy(dst_hbm.at[pl.ds(s * CPT, CPT)], dst_v)

    def body(j, carry):
        pltpu.sync_copy(ones_v, sdeg.at[dst_v.at[j]], add=True)
        return carry
    lax.fori_loop(0, CPT, body, 0)
    plsc.subcore_barrier()

    @pl.when(c == 0)
    def _():
        pltpu.sync_copy(sdeg.at[pl.ds(s * STRIPE, STRIPE)],
                        deg_hbm.at[pl.ds(s * STRIPE, STRIPE)])


@functools.partial(
    pl.kernel,
    out_type=jax.ShapeDtypeStruct((4, NP, QD), jnp.float32),
    mesh=_mesh,
    compiler_params=pltpu.CompilerParams(use_tc_tiling_on_sc=False),
    scratch_types=[
        pltpu.VMEM((CPT, 128), jnp.int32),
        pltpu.VMEM((CPT, 128), jnp.int32),
        pltpu.VMEM((RD, 128, QD), jnp.float32),
        pltpu.VMEM_SHARED((NP, QD), jnp.float32),
        pltpu.SemaphoreType.DMA((RD,)),
        pltpu.SemaphoreType.DMA((RD,)),
    ],
)
def _agg_kernel(sflat_hbm, src_hbm, dst_hbm, zeros_hbm, out_hbm,
                src_v, dst_v, rows, sout, gsem, ssem):
    c = lax.axis_index("c")
    s = lax.axis_index("s")

    pltpu.sync_copy(dst_hbm.at[pl.ds(s * CPT, CPT)], dst_v)

    def pass_body(p, pcarry):
        q = 2 * c + p
        pltpu.sync_copy(zeros_hbm.at[pl.ds(s * STRIPE, STRIPE)],
                        sout.at[pl.ds(s * STRIPE, STRIPE)])
        pltpu.sync_copy(src_hbm.at[q, pl.ds(s * CPT, CPT)], src_v)
        plsc.subcore_barrier()

        def body(j, carry):
            b = lax.rem(j, RD)

            @pl.when(jnp.logical_and(j >= RD, j - RD < CPT))
            def _():
                pltpu.make_async_copy(
                    rows.at[b], sout.at[dst_v.at[lax.rem(j, CPT)]],
                    ssem.at[b]).wait()

            @pl.when(j < CPT)
            def _():
                pltpu.async_copy(sflat_hbm.at[src_v.at[j]],
                                 rows.at[b], gsem.at[b])

            jj = j - LAG
            bb = lax.rem(j + RD - LAG, RD)

            @pl.when(jnp.logical_and(jj >= 0, jj < CPT))
            def _():
                pltpu.make_async_copy(sflat_hbm.at[pl.ds(0, 128)],
                                      rows.at[bb], gsem.at[bb]).wait()
                pltpu.async_copy(rows.at[bb], sout.at[dst_v.at[jj]],
                                 ssem.at[bb], add=True)
            return carry
        lax.fori_loop(0, CPT + RD, body, 0)

        plsc.subcore_barrier()
        pltpu.sync_copy(sout.at[pl.ds(s * STRIPE, STRIPE)],
                        out_hbm.at[q, pl.ds(s * STRIPE, STRIPE)])
        return pcarry
    lax.fori_loop(0, 2, pass_body, 0)


def _mm_halves(h0, h1, w_ref):
    outs = []
    for cp in range(2):
        u = jnp.dot(h0, w_ref[0, :, cp, :], preferred_element_type=jnp.float32)
        u = u + jnp.dot(h1, w_ref[1, :, cp, :],
                        preferred_element_type=jnp.float32)
        outs.append(u)
    return outs


def _store_quarters(o_ref, u0, u1, dinv):
    v0 = u0 * dinv
    v1 = u1 * dinv
    o_ref[0] = v0[:, :QD]
    o_ref[1] = v0[:, QD:]
    o_ref[2] = v1[:, :QD]
    o_ref[3] = v1[:, QD:]


def _a1_body(x_ref, deg_ref, w_ref, o_ref):
    dinv = lax.rsqrt(deg_ref[0, 0, :] + 1.0)[:, None]
    u0, u1 = _mm_halves(x_ref[:, :HD], x_ref[:, HD:], w_ref)
    _store_quarters(o_ref, u0, u1, dinv)


def _halves(q_ref):
    a = jnp.concatenate([q_ref[0], q_ref[1]], axis=1)
    b = jnp.concatenate([q_ref[2], q_ref[3]], axis=1)
    return a, b


def _a_body(e_ref, s_ref, deg_ref, w_ref, b_ref, o_ref):
    dinv = lax.rsqrt(deg_ref[0, 0, :] + 1.0)[:, None]
    e0, e1 = _halves(e_ref)
    s0, s1 = _halves(s_ref)
    h0 = jnp.maximum(dinv * (e0 + s0) + b_ref[0], 0.0)
    h1 = jnp.maximum(dinv * (e1 + s1) + b_ref[1], 0.0)
    u0, u1 = _mm_halves(h0, h1, w_ref)
    _store_quarters(o_ref, u0, u1, dinv)


def _f_body(e_ref, s_ref, deg_ref, b_ref, batch_ref,
            l1w_ref, l1b_ref, l2w_ref, l2b_ref, o_ref,
            acc0, acc1, cnt):
    i = pl.program_id(0)

    @pl.when(i == 0)
    def _():
        acc0[:] = jnp.zeros((NUM_GRAPHS, HD), jnp.float32)
        acc1[:] = jnp.zeros((NUM_GRAPHS, HD), jnp.float32)
        cnt[:] = jnp.zeros((NUM_GRAPHS, HD), jnp.float32)

    dinv = lax.rsqrt(deg_ref[0, 0, :] + 1.0)[:, None]
    e0, e1 = _halves(e_ref)
    s0, s1 = _halves(s_ref)
    h0 = jnp.maximum(dinv * (e0 + s0) + b_ref[0], 0.0)
    h1 = jnp.maximum(dinv * (e1 + s1) + b_ref[1], 0.0)
    gids = lax.broadcasted_iota(jnp.int32, (NUM_GRAPHS, R), 0)
    pt = (gids == batch_ref[0, 0, :][None, :]).astype(jnp.float32)
    acc0[:] += jnp.dot(pt, h0, preferred_element_type=jnp.float32)
    acc1[:] += jnp.dot(pt, h1, preferred_element_type=jnp.float32)
    cnt[:] += jnp.broadcast_to(jnp.sum(pt, axis=1, keepdims=True),
                               (NUM_GRAPHS, HD))

    @pl.when(i == NBLK - 1)
    def _():
        cts = jnp.maximum(cnt[:, 0:1], 1.0)
        p0 = acc0[:] / cts
        p1 = acc1[:] / cts
        z = jnp.dot(p0, l1w_ref[0], preferred_element_type=jnp.float32)
        z = z + jnp.dot(p1, l1w_ref[1], preferred_element_type=jnp.float32)
        z = jnp.maximum(z + l1b_ref[0, :][None, :], 0.0)
        o_ref[:] = (jnp.dot(z, l2w_ref[:], preferred_element_type=jnp.float32)
                    + l2b_ref[0, :][None, :])


def _tc_a1(xp, deg3, w):
    return pl.pallas_call(
        _a1_body,
        grid=(NBLK,),
        in_specs=[
            pl.BlockSpec((R, D), lambda i: (i, 0)),
            pl.BlockSpec((1, 1, R), lambda i: (i, 0, 0)),
            pl.BlockSpec((2, HD, 2, HD), lambda i: (0, 0, 0, 0)),
        ],
        out_specs=pl.BlockSpec((4, R, QD), lambda i: (0, i, 0)),
        out_shape=jax.ShapeDtypeStruct((4, NP, QD), jnp.float32),
    )(xp, deg3, w)


def _tc_a(e, sv, deg3, w, b):
    return pl.pallas_call(
        _a_body,
        grid=(NBLK,),
        in_specs=[
            pl.BlockSpec((4, R, QD), lambda i: (0, i, 0)),
            pl.BlockSpec((4, R, QD), lambda i: (0, i, 0)),
            pl.BlockSpec((1, 1, R), lambda i: (i, 0, 0)),
            pl.BlockSpec((2, HD, 2, HD), lambda i: (0, 0, 0, 0)),
            pl.BlockSpec((2, HD), lambda i: (0, 0)),
        ],
        out_specs=pl.BlockSpec((4, R, QD), lambda i: (0, i, 0)),
        out_shape=jax.ShapeDtypeStruct((4, NP, QD), jnp.float32),
    )(e, sv, deg3, w, b)


def _tc_final(e, sv, deg3, b, batch3, l1w, l1b, l2w, l2b):
    return pl.pallas_call(
        _f_body,
        grid=(NBLK,),
        in_specs=[
            pl.BlockSpec((4, R, QD), lambda i: (0, i, 0)),
            pl.BlockSpec((4, R, QD), lambda i: (0, i, 0)),
            pl.BlockSpec((1, 1, R), lambda i: (i, 0, 0)),
            pl.BlockSpec((2, HD), lambda i: (0, 0)),
            pl.BlockSpec((1, 1, R), lambda i: (i, 0, 0)),
            pl.BlockSpec((2, HD, D), lambda i: (0, 0, 0)),
            pl.BlockSpec((1, D), lambda i: (0, 0)),
            pl.BlockSpec((D, 10), lambda i: (0, 0)),
            pl.BlockSpec((1, 10), lambda i: (0, 0)),
        ],
        out_specs=pl.BlockSpec((NUM_GRAPHS, 10), lambda i: (0, 0)),
        out_shape=jax.ShapeDtypeStruct((NUM_GRAPHS, 10), jnp.float32),
        scratch_shapes=[
            pltpu.VMEM((NUM_GRAPHS, HD), jnp.float32),
            pltpu.VMEM((NUM_GRAPHS, HD), jnp.float32),
            pltpu.VMEM((NUM_GRAPHS, HD), jnp.float32),
        ],
    )(e, sv, deg3, b, batch3, l1w, l1b, l2w, l2b)


def kernel(x, edge_index, batch, conv_W, conv_b, lin1_W, lin1_b, lin2_W,
           lin2_b):
    src = edge_index[0].astype(jnp.int32)
    dst = edge_index[1].astype(jnp.int32)
    pad = jnp.full((EP - E,), N, jnp.int32)
    src_p = jnp.concatenate([src, pad])
    dst_p = jnp.concatenate([dst, pad])
    src4 = jnp.stack([src_p + q * NP for q in range(4)]).reshape(4, CT, 128)
    dst2 = dst_p.reshape(CT, 128)

    xp = jnp.pad(x, ((0, NP - N), (0, 0)))
    batch3 = jnp.concatenate(
        [batch.astype(jnp.int32),
         jnp.full((NP - N,), NUM_GRAPHS, jnp.int32)]).reshape(NBLK, 1, R)

    w = conv_W.reshape(3, 2, HD, 2, HD)
    b = conv_b.reshape(3, 2, HD)
    l1w = lin1_W.reshape(2, HD, D)
    l1b = lin1_b.reshape(1, D)
    l2b = lin2_b.reshape(1, 10)

    deg = _deg_kernel(dst2)
    deg3 = deg.reshape(NBLK, 1, R)

    s1 = _tc_a1(xp, deg3, w[0])
    zeros_h = jnp.zeros((NP, QD), jnp.float32)
    e1 = _agg_kernel(s1.reshape(4 * NP, QD), src4, dst2, zeros_h)
    s2 = _tc_a(e1, s1, deg3, w[1], b[0])
    e2 = _agg_kernel(s2.reshape(4 * NP, QD), src4, dst2, zeros_h)
    s3 = _tc_a(e2, s2, deg3, w[2], b[1])
    e3 = _agg_kernel(s3.reshape(4 * NP, QD), src4, dst2, zeros_h)

    return _tc_final(e3, s3, deg3, b[2], batch3, l1w, l1b, lin2_W, l2b)


# speedup vs baseline: 1.1905x; 1.0237x over previous
import functools

import jax
import jax.numpy as jnp
from jax import lax
from jax.experimental import pallas as pl
from jax.experimental.pallas import tpu as pltpu
from jax.experimental.pallas import tpu_sc as plsc

N = 10000
NP = 10112
D = 256
E = 160000
EP = 163840
CT = EP // 128
NS = 16
CPH = CT // 2
CPT = CPH // NS
DCPT = CT // NS
RD = 2
LAG = 1
STRIPE = NP // NS
NUM_GRAPHS = 64
R = 128
NBLK = NP // R

_mesh = plsc.VectorSubcoreMesh(core_axis_name="c", subcore_axis_name="s")


NPD = 10240
STRIPED = NPD // NS


@functools.partial(
    pl.kernel,
    out_type=jax.ShapeDtypeStruct((NPD,), jnp.float32),
    mesh=_mesh,
    scratch_types=[
        pltpu.VMEM((DCPT, 128), jnp.int32),
        pltpu.VMEM((128,), jnp.float32),
        pltpu.VMEM((STRIPED,), jnp.float32),
        pltpu.VMEM_SHARED((NPD,), jnp.float32),
    ],
)
def _deg_kernel(dst_hbm, deg_hbm, dst_v, ones_v, zb_v, sdeg):
    c = lax.axis_index("c")
    s = lax.axis_index("s")
    for k in range(8):
        ones_v[pl.ds(k * 16, 16)] = jnp.ones((16,), jnp.float32)

    def zb(i, carry):
        zb_v[pl.ds(i * 16, 16)] = jnp.zeros((16,), jnp.float32)
        return carry
    lax.fori_loop(0, STRIPED // 16, zb, 0)
    pltpu.sync_copy(zb_v, sdeg.at[pl.ds(s * STRIPED, STRIPED)])
    plsc.subcore_barrier()

    pltpu.sync_copy(dst_hbm.at[pl.ds(s * DCPT, DCPT)], dst_v)

    def body(j, carry):
        pltpu.sync_copy(ones_v, sdeg.at[dst_v.at[j]], add=True)
        return carry
    lax.fori_loop(0, DCPT, body, 0)
    plsc.subcore_barrier()

    @pl.when(c == 0)
    def _():
        pltpu.sync_copy(sdeg.at[pl.ds(s * STRIPED, STRIPED)],
                        deg_hbm.at[pl.ds(s * STRIPED, STRIPED)])


@functools.partial(
    pl.kernel,
    out_type=jax.ShapeDtypeStruct((2, NP, D), jnp.bfloat16),
    mesh=_mesh,
    compiler_params=pltpu.CompilerParams(use_tc_tiling_on_sc=False),
    scratch_types=[
        pltpu.VMEM((CPT, 128), jnp.int32),
        pltpu.VMEM((CPT, 128), jnp.int32),
        pltpu.VMEM((RD, 128, D), jnp.bfloat16),
        pltpu.VMEM_SHARED((NP, D), jnp.bfloat16),
        pltpu.SemaphoreType.DMA((RD,)),
        pltpu.SemaphoreType.DMA((RD,)),
    ],
)
def _agg_kernel(s_hbm, src_hbm, dst_hbm, zeros_hbm, out_hbm,
                src_v, dst_v, rows, sout, gsem, ssem):
    c = lax.axis_index("c")
    s = lax.axis_index("s")

    pltpu.sync_copy(dst_hbm.at[c, pl.ds(s * CPT, CPT)], dst_v)
    pltpu.sync_copy(zeros_hbm.at[pl.ds(s * STRIPE, STRIPE)],
                    sout.at[pl.ds(s * STRIPE, STRIPE)])
    pltpu.sync_copy(src_hbm.at[c, pl.ds(s * CPT, CPT)], src_v)
    plsc.subcore_barrier()

    def body(j, carry):
        b = lax.rem(j, RD)

        @pl.when(jnp.logical_and(j >= RD, j - RD < CPT))
        def _():
            pltpu.make_async_copy(
                rows.at[b], sout.at[dst_v.at[lax.rem(j, CPT)]],
                ssem.at[b]).wait()

        @pl.when(j < CPT)
        def _():
            pltpu.async_copy(s_hbm.at[src_v.at[j]], rows.at[b], gsem.at[b])

        jj = j - LAG
        bb = lax.rem(j + RD - LAG, RD)

        @pl.when(jnp.logical_and(jj >= 0, jj < CPT))
        def _():
            pltpu.make_async_copy(s_hbm.at[pl.ds(0, 128)],
                                  rows.at[bb], gsem.at[bb]).wait()
            pltpu.async_copy(rows.at[bb], sout.at[dst_v.at[jj]],
                             ssem.at[bb], add=True)
        return carry
    lax.fori_loop(0, CPT + RD, body, 0)

    plsc.subcore_barrier()
    pltpu.sync_copy(sout.at[pl.ds(s * STRIPE, STRIPE)],
                    out_hbm.at[c, pl.ds(s * STRIPE, STRIPE)])


def _a1_body(x_ref, deg_ref, w_ref, o_ref):
    dinv = lax.rsqrt(deg_ref[0, 0, :] + 1.0)[:, None]
    u = jnp.dot(x_ref[:], w_ref[:], preferred_element_type=jnp.float32)
    o_ref[:] = (u * dinv).astype(jnp.bfloat16)


def _h_block(e_ref, s_ref, deg_ref, b_ref):
    dinv = lax.rsqrt(deg_ref[0, 0, :] + 1.0)[:, None]
    t = (e_ref[0].astype(jnp.float32) + e_ref[1].astype(jnp.float32)
         + s_ref[0].astype(jnp.float32))
    return jnp.maximum(dinv * t + b_ref[0, :][None, :], 0.0), dinv


def _a_body(e_ref, s_ref, deg_ref, w_ref, b_ref, o_ref):
    h, dinv = _h_block(e_ref, s_ref, deg_ref, b_ref)
    u = jnp.dot(h, w_ref[:], preferred_element_type=jnp.float32)
    o_ref[:] = (u * dinv).astype(jnp.bfloat16)


def _f_body(e_ref, s_ref, deg_ref, b_ref, batch_ref,
            l1w_ref, l1b_ref, l2w_ref, l2b_ref, o_ref, acc, cnt):
    i = pl.program_id(0)

    @pl.when(i == 0)
    def _():
        acc[:] = jnp.zeros((NUM_GRAPHS, D), jnp.float32)
        cnt[:] = jnp.zeros((NUM_GRAPHS, R), jnp.float32)

    h, _ = _h_block(e_ref, s_ref, deg_ref, b_ref)
    gids = lax.broadcasted_iota(jnp.int32, (NUM_GRAPHS, R), 0)
    pt = (gids == batch_ref[0, 0, :][None, :]).astype(jnp.float32)
    acc[:] += jnp.dot(pt, h, preferred_element_type=jnp.float32)
    cnt[:] += jnp.broadcast_to(jnp.sum(pt, axis=1, keepdims=True),
                               (NUM_GRAPHS, R))

    @pl.when(i == NBLK - 1)
    def _():
        cts = jnp.maximum(cnt[:, 0:1], 1.0)
        p = acc[:] / cts
        z = jnp.maximum(
            jnp.dot(p, l1w_ref[:], preferred_element_type=jnp.float32)
            + l1b_ref[0, :][None, :], 0.0)
        o_ref[:] = (jnp.dot(z, l2w_ref[:], preferred_element_type=jnp.float32)
                    + l2b_ref[0, :][None, :])


def _tc_a1(xp, deg3, w):
    return pl.pallas_call(
        _a1_body,
        grid=(NBLK,),
        in_specs=[
            pl.BlockSpec((R, D), lambda i: (i, 0)),
            pl.BlockSpec((1, 1, R), lambda i: (i, 0, 0)),
            pl.BlockSpec((D, D), lambda i: (0, 0)),
        ],
        out_specs=pl.BlockSpec((R, D), lambda i: (i, 0)),
        out_shape=jax.ShapeDtypeStruct((NP, D), jnp.bfloat16),
    )(xp, deg3, w)


def _tc_a(e, sv, deg3, w, b):
    return pl.pallas_call(
        _a_body,
        grid=(NBLK,),
        in_specs=[
            pl.BlockSpec((2, R, D), lambda i: (0, i, 0)),
            pl.BlockSpec((1, R, D), lambda i: (0, i, 0)),
            pl.BlockSpec((1, 1, R), lambda i: (i, 0, 0)),
            pl.BlockSpec((D, D), lambda i: (0, 0)),
            pl.BlockSpec((1, D), lambda i: (0, 0)),
        ],
        out_specs=pl.BlockSpec((R, D), lambda i: (i, 0)),
        out_shape=jax.ShapeDtypeStruct((NP, D), jnp.bfloat16),
    )(e, sv.reshape(1, NP, D), deg3, w, b)


def _tc_final(e, sv, deg3, b, batch3, l1w, l1b, l2w, l2b):
    return pl.pallas_call(
        _f_body,
        grid=(NBLK,),
        in_specs=[
            pl.BlockSpec((2, R, D), lambda i: (0, i, 0)),
            pl.BlockSpec((1, R, D), lambda i: (0, i, 0)),
            pl.BlockSpec((1, 1, R), lambda i: (i, 0, 0)),
            pl.BlockSpec((1, D), lambda i: (0, 0)),
            pl.BlockSpec((1, 1, R), lambda i: (i, 0, 0)),
            pl.BlockSpec((D, D), lambda i: (0, 0)),
            pl.BlockSpec((1, D), lambda i: (0, 0)),
            pl.BlockSpec((D, 10), lambda i: (0, 0)),
            pl.BlockSpec((1, 10), lambda i: (0, 0)),
        ],
        out_specs=pl.BlockSpec((NUM_GRAPHS, 10), lambda i: (0, 0)),
        out_shape=jax.ShapeDtypeStruct((NUM_GRAPHS, 10), jnp.float32),
        scratch_shapes=[
            pltpu.VMEM((NUM_GRAPHS, D), jnp.float32),
            pltpu.VMEM((NUM_GRAPHS, R), jnp.float32),
        ],
    )(e, sv.reshape(1, NP, D), deg3, b, batch3, l1w, l1b, l2w, l2b)


def kernel(x, edge_index, batch, conv_W, conv_b, lin1_W, lin1_b, lin2_W,
           lin2_b):
    src = edge_index[0].astype(jnp.int32)
    dst = edge_index[1].astype(jnp.int32)
    pad = jnp.full((EP - E,), N, jnp.int32)
    src2 = jnp.concatenate([src, pad]).reshape(2, CPH, 128)
    dst_p = jnp.concatenate([dst, pad])
    dst2 = dst_p.reshape(2, CPH, 128)
    dstd = dst_p.reshape(CT, 128)

    xp = jnp.pad(x, ((0, NP - N), (0, 0)))
    batch3 = jnp.concatenate(
        [batch.astype(jnp.int32),
         jnp.full((NP - N,), NUM_GRAPHS, jnp.int32)]).reshape(NBLK, 1, R)

    b = conv_b.reshape(3, 1, D)
    l1b = lin1_b.reshape(1, D)
    l2b = lin2_b.reshape(1, 10)

    deg = _deg_kernel(dstd)
    deg3 = deg[:NP].reshape(NBLK, 1, R)

    zeros_h = jnp.zeros((NP, D), jnp.bfloat16)
    s1 = _tc_a1(xp, deg3, conv_W[0])
    e1 = _agg_kernel(s1, src2, dst2, zeros_h)
    s2 = _tc_a(e1, s1, deg3, conv_W[1], b[0])
    e2 = _agg_kernel(s2, src2, dst2, zeros_h)
    s3 = _tc_a(e2, s2, deg3, conv_W[2], b[1])
    e3 = _agg_kernel(s3, src2, dst2, zeros_h)

    return _tc_final(e3, s3, deg3, b[2], batch3, lin1_W, l1b, lin2_W, l2b)


# speedup vs baseline: 1.7900x; 1.5036x over previous
import functools

import jax
import jax.numpy as jnp
from jax import lax
from jax.experimental import pallas as pl
from jax.experimental.pallas import tpu as pltpu
from jax.experimental.pallas import tpu_sc as plsc

N = 10000
NP = 10240
D = 256
HD = 128
QD = 64
E = 160000
EP = 163840
CT = EP // 128
NS = 16
CPT = CT // NS
RD = 8
LAG = 4
STRIPE = NP // NS
NUM_GRAPHS = 64
R = 256
NBLK = NP // R

_mesh = plsc.VectorSubcoreMesh(core_axis_name="c", subcore_axis_name="s")


@functools.partial(
    pl.kernel,
    out_type=jax.ShapeDtypeStruct((NP,), jnp.float32),
    mesh=_mesh,
    scratch_types=[
        pltpu.VMEM((CPT, 128), jnp.int32),
        pltpu.VMEM((128,), jnp.float32),
        pltpu.VMEM((STRIPE,), jnp.float32),
        pltpu.VMEM_SHARED((NP,), jnp.float32),
    ],
)
def _deg_kernel(dst_hbm, deg_hbm, dst_v, ones_v, zb_v, sdeg):
    c = lax.axis_index("c")
    s = lax.axis_index("s")
    for k in range(8):
        ones_v[pl.ds(k * 16, 16)] = jnp.ones((16,), jnp.float32)

    def zb(i, carry):
        zb_v[pl.ds(i * 16, 16)] = jnp.zeros((16,), jnp.float32)
        return carry
    lax.fori_loop(0, STRIPE // 16, zb, 0)
    pltpu.sync_copy(zb_v, sdeg.at[pl.ds(s * STRIPE, STRIPE)])
    plsc.subcore_barrier()

    pltpu.sync_copy(dst_hbm.at[pl.ds(s * CPT, CPT)], dst_v)

    def body(j, carry):
        pltpu.sync_copy(ones_v, sdeg.at[dst_v.at[j]], add=True)
        return carry
    lax.fori_loop(0, CPT, body, 0)
    plsc.subcore_barrier()

    @pl.when(c == 0)
    def _():
        pltpu.sync_copy(sdeg.at[pl.ds(s * STRIPE, STRIPE)],
                        deg_hbm.at[pl.ds(s * STRIPE, STRIPE)])


@functools.partial(
    pl.kernel,
    out_type=jax.ShapeDtypeStruct((2, NP, HD), jnp.bfloat16),
    mesh=_mesh,
    compiler_params=pltpu.CompilerParams(use_tc_tiling_on_sc=False),
    scratch_types=[
        pltpu.VMEM((CPT, 128), jnp.int32),
        pltpu.VMEM((CPT, 128), jnp.int32),
        pltpu.VMEM((RD, 128, HD), jnp.bfloat16),
        pltpu.VMEM_SHARED((NP, HD), jnp.bfloat16),
        pltpu.SemaphoreType.DMA((RD,)),
        pltpu.SemaphoreType.DMA((RD,)),
    ],
)
def _agg_kernel(sflat_hbm, src_hbm, dst_hbm, zeros_hbm, out_hbm,
                src_v, dst_v, rows, sout, gsem, ssem):
    c = lax.axis_index("c")
    s = lax.axis_index("s")

    pltpu.sync_copy(dst_hbm.at[pl.ds(s * CPT, CPT)], dst_v)
    pltpu.sync_copy(zeros_hbm.at[pl.ds(s * STRIPE, STRIPE)],
                    sout.at[pl.ds(s * STRIPE, STRIPE)])
    pltpu.sync_copy(src_hbm.at[c, pl.ds(s * CPT, CPT)], src_v)
    plsc.subcore_barrier()

    def body(j, carry):
        b = lax.rem(j, RD)

        @pl.when(jnp.logical_and(j >= RD, j - RD < CPT))
        def _():
            pltpu.make_async_copy(
                rows.at[b], sout.at[dst_v.at[lax.rem(j, CPT)]],
                ssem.at[b]).wait()

        @pl.when(j < CPT)
        def _():
            pltpu.async_copy(sflat_hbm.at[src_v.at[j]],
                             rows.at[b], gsem.at[b])

        jj = j - LAG
        bb = lax.rem(j + RD - LAG, RD)

        @pl.when(jnp.logical_and(jj >= 0, jj < CPT))
        def _():
            pltpu.make_async_copy(sflat_hbm.at[pl.ds(0, 128)],
                                  rows.at[bb], gsem.at[bb]).wait()
            pltpu.async_copy(rows.at[bb], sout.at[dst_v.at[jj]],
                             ssem.at[bb], add=True)
        return carry
    lax.fori_loop(0, CPT + RD, body, 0)

    plsc.subcore_barrier()
    pltpu.sync_copy(sout.at[pl.ds(s * STRIPE, STRIPE)],
                    out_hbm.at[c, pl.ds(s * STRIPE, STRIPE)])


def _mm_halves(h0, h1, w_ref):
    outs = []
    for cp in range(2):
        u = jnp.dot(h0, w_ref[0, :, cp, :], preferred_element_type=jnp.float32)
        u = u + jnp.dot(h1, w_ref[1, :, cp, :],
                        preferred_element_type=jnp.float32)
        outs.append(u)
    return outs


def _store_halves(o_ref, u0, u1, dinv):
    o_ref[0] = (u0 * dinv).astype(jnp.bfloat16)
    o_ref[1] = (u1 * dinv).astype(jnp.bfloat16)


def _a1_body(x_ref, deg_ref, w_ref, o_ref):
    dinv = lax.rsqrt(deg_ref[0, 0, :] + 1.0)[:, None]
    u0, u1 = _mm_halves(x_ref[:, :HD], x_ref[:, HD:], w_ref)
    _store_halves(o_ref, u0, u1, dinv)


def _a_body(e_ref, s_ref, deg_ref, w_ref, b_ref, o_ref):
    dinv = lax.rsqrt(deg_ref[0, 0, :] + 1.0)[:, None]
    e0 = e_ref[0].astype(jnp.float32) + s_ref[0].astype(jnp.float32)
    e1 = e_ref[1].astype(jnp.float32) + s_ref[1].astype(jnp.float32)
    h0 = jnp.maximum(dinv * e0 + b_ref[0], 0.0)
    h1 = jnp.maximum(dinv * e1 + b_ref[1], 0.0)
    u0, u1 = _mm_halves(h0, h1, w_ref)
    _store_halves(o_ref, u0, u1, dinv)


def _f_body(e_ref, s_ref, deg_ref, b_ref, batch_ref,
            l1w_ref, l1b_ref, l2w_ref, l2b_ref, o_ref,
            acc0, acc1, cnt):
    i = pl.program_id(0)

    @pl.when(i == 0)
    def _():
        acc0[:] = jnp.zeros((NUM_GRAPHS, HD), jnp.float32)
        acc1[:] = jnp.zeros((NUM_GRAPHS, HD), jnp.float32)
        cnt[:] = jnp.zeros((NUM_GRAPHS, HD), jnp.float32)

    dinv = lax.rsqrt(deg_ref[0, 0, :] + 1.0)[:, None]
    e0 = e_ref[0].astype(jnp.float32) + s_ref[0].astype(jnp.float32)
    e1 = e_ref[1].astype(jnp.float32) + s_ref[1].astype(jnp.float32)
    h0 = jnp.maximum(dinv * e0 + b_ref[0], 0.0)
    h1 = jnp.maximum(dinv * e1 + b_ref[1], 0.0)
    gids = lax.broadcasted_iota(jnp.int32, (NUM_GRAPHS, R), 0)
    pt = (gids == batch_ref[0, 0, :][None, :]).astype(jnp.float32)
    acc0[:] += jnp.dot(pt, h0, preferred_element_type=jnp.float32)
    acc1[:] += jnp.dot(pt, h1, preferred_element_type=jnp.float32)
    cnt[:] += jnp.broadcast_to(jnp.sum(pt, axis=1, keepdims=True),
                               (NUM_GRAPHS, HD))

    @pl.when(i == NBLK - 1)
    def _():
        cts = jnp.maximum(cnt[:, 0:1], 1.0)
        p0 = acc0[:] / cts
        p1 = acc1[:] / cts
        z = jnp.dot(p0, l1w_ref[0], preferred_element_type=jnp.float32)
        z = z + jnp.dot(p1, l1w_ref[1], preferred_element_type=jnp.float32)
        z = jnp.maximum(z + l1b_ref[0, :][None, :], 0.0)
        o_ref[:] = (jnp.dot(z, l2w_ref[:], preferred_element_type=jnp.float32)
                    + l2b_ref[0, :][None, :])


def _tc_a1(xp, deg3, w):
    return pl.pallas_call(
        _a1_body,
        grid=(NBLK,),
        in_specs=[
            pl.BlockSpec((R, D), lambda i: (i, 0)),
            pl.BlockSpec((1, 1, R), lambda i: (i, 0, 0)),
            pl.BlockSpec((2, HD, 2, HD), lambda i: (0, 0, 0, 0)),
        ],
        out_specs=pl.BlockSpec((2, R, HD), lambda i: (0, i, 0)),
        out_shape=jax.ShapeDtypeStruct((2, NP, HD), jnp.bfloat16),
    )(xp, deg3, w)


def _tc_a(e, sv, deg3, w, b):
    return pl.pallas_call(
        _a_body,
        grid=(NBLK,),
        in_specs=[
            pl.BlockSpec((2, R, HD), lambda i: (0, i, 0)),
            pl.BlockSpec((2, R, HD), lambda i: (0, i, 0)),
            pl.BlockSpec((1, 1, R), lambda i: (i, 0, 0)),
            pl.BlockSpec((2, HD, 2, HD), lambda i: (0, 0, 0, 0)),
            pl.BlockSpec((2, HD), lambda i: (0, 0)),
        ],
        out_specs=pl.BlockSpec((2, R, HD), lambda i: (0, i, 0)),
        out_shape=jax.ShapeDtypeStruct((2, NP, HD), jnp.bfloat16),
    )(e, sv, deg3, w, b)


def _tc_final(e, sv, deg3, b, batch3, l1w, l1b, l2w, l2b):
    return pl.pallas_call(
        _f_body,
        grid=(NBLK,),
        in_specs=[
            pl.BlockSpec((2, R, HD), lambda i: (0, i, 0)),
            pl.BlockSpec((2, R, HD), lambda i: (0, i, 0)),
            pl.BlockSpec((1, 1, R), lambda i: (i, 0, 0)),
            pl.BlockSpec((2, HD), lambda i: (0, 0)),
            pl.BlockSpec((1, 1, R), lambda i: (i, 0, 0)),
            pl.BlockSpec((2, HD, D), lambda i: (0, 0, 0)),
            pl.BlockSpec((1, D), lambda i: (0, 0)),
            pl.BlockSpec((D, 10), lambda i: (0, 0)),
            pl.BlockSpec((1, 10), lambda i: (0, 0)),
        ],
        out_specs=pl.BlockSpec((NUM_GRAPHS, 10), lambda i: (0, 0)),
        out_shape=jax.ShapeDtypeStruct((NUM_GRAPHS, 10), jnp.float32),
        scratch_shapes=[
            pltpu.VMEM((NUM_GRAPHS, HD), jnp.float32),
            pltpu.VMEM((NUM_GRAPHS, HD), jnp.float32),
            pltpu.VMEM((NUM_GRAPHS, HD), jnp.float32),
        ],
    )(e, sv, deg3, b, batch3, l1w, l1b, l2w, l2b)


def kernel(x, edge_index, batch, conv_W, conv_b, lin1_W, lin1_b, lin2_W,
           lin2_b):
    src = edge_index[0].astype(jnp.int32)
    dst = edge_index[1].astype(jnp.int32)
    pad = jnp.full((EP - E,), N, jnp.int32)
    src_p = jnp.concatenate([src, pad])
    dst_p = jnp.concatenate([dst, pad])
    src2 = jnp.stack([src_p, src_p + NP]).reshape(2, CT, 128)
    dst2 = dst_p.reshape(CT, 128)

    xp = jnp.pad(x, ((0, NP - N), (0, 0)))
    batch3 = jnp.concatenate(
        [batch.astype(jnp.int32),
         jnp.full((NP - N,), NUM_GRAPHS, jnp.int32)]).reshape(NBLK, 1, R)

    w = conv_W.reshape(3, 2, HD, 2, HD)
    b = conv_b.reshape(3, 2, HD)
    l1w = lin1_W.reshape(2, HD, D)
    l1b = lin1_b.reshape(1, D)
    l2b = lin2_b.reshape(1, 10)

    deg = _deg_kernel(dst2)
    deg3 = deg.reshape(NBLK, 1, R)

    s1 = _tc_a1(xp, deg3, w[0])
    zeros_h = jnp.zeros((NP, HD), jnp.bfloat16)
    e1 = _agg_kernel(s1.reshape(2 * NP, HD), src2, dst2, zeros_h)
    s2 = _tc_a(e1, s1, deg3, w[1], b[0])
    e2 = _agg_kernel(s2.reshape(2 * NP, HD), src2, dst2, zeros_h)
    s3 = _tc_a(e2, s2, deg3, w[2], b[1])
    e3 = _agg_kernel(s3.reshape(2 * NP, HD), src2, dst2, zeros_h)

    return _tc_final(e3, s3, deg3, b[2], batch3, l1w, l1b, lin2_W, l2b)
